# baseline (speedup 1.0000x reference)
import functools

import jax
import jax.numpy as jnp
from jax import lax
from jax.experimental import pallas as pl
from jax.experimental.pallas import tpu as pltpu
from jax.experimental.pallas import tpu_sc as plsc

L = 16
NS = 16
NC = 2
KC = 32
B = 512
G = 128
ZR = 128


def _sc_spmm(col2, row2, vals, y_flat, *, M, N, NB):
    mesh = plsc.VectorSubcoreMesh(core_axis_name="c", subcore_axis_name="s")
    nsub = B // G

    @functools.partial(
        pl.kernel,
        out_type=jax.ShapeDtypeStruct((8 * M, KC), jnp.float32),
        mesh=mesh,
        scratch_types=[
            pltpu.VMEM_SHARED((M, KC), jnp.float32),
            pltpu.VMEM((2, nsub, G), jnp.int32),
            pltpu.VMEM((2, nsub, G), jnp.int32),
            pltpu.VMEM((2, B, L), jnp.float32),
            pltpu.VMEM((2, B, KC), jnp.float32),
            pltpu.VMEM((ZR, KC), jnp.float32),
            pltpu.SemaphoreType.DMA,
            pltpu.SemaphoreType.DMA,
        ],
        compiler_params=pltpu.CompilerParams(use_tc_tiling_on_sc=False),
    )
    def kfn(col_hbm, row_hbm, vals_hbm, y_hbm, out_hbm,
            acc, cidx, ridx, vals_v, rows, zbuf, gsem, ssem):
        c = lax.axis_index("c")
        s = lax.axis_index("s")
        zero16 = jnp.zeros((L,), jnp.float32)

        def zb_body(i, carry):
            for j in range(KC // L):
                zbuf[i, pl.ds(L * j, L)] = zero16
            return carry

        lax.fori_loop(0, ZR, zb_body, 0)

        rows_per_tile = M // NS

        def stage(b, p, off):
            base = s * NB + b
            pltpu.sync_copy(col_hbm.at[pl.ds(base * nsub, nsub)], cidx.at[p])
            for r in range(nsub):
                for l in range(G // L):
                    cidx[p, r, pl.ds(L * l, L)] = (
                        cidx[p, r, pl.ds(L * l, L)] + off)
            for g in range(nsub):
                pltpu.async_copy(y_hbm.at[cidx.at[p, g]],
                                 rows.at[p, pl.ds(g * G, G)], gsem)
            pltpu.sync_copy(row_hbm.at[pl.ds(base * nsub, nsub)], ridx.at[p])
            pltpu.sync_copy(vals_hbm.at[pl.ds(base * B, B)], vals_v.at[p])

        def multiply(p):
            @plsc.parallel_loop(0, B, unroll=8)
            def mul_body(i):
                v = vals_v[p, i]
                for j in range(KC // L):
                    rows[p, i, pl.ds(L * j, L)] = (
                        rows[p, i, pl.ds(L * j, L)] * v)

        def drain(sem, p):
            pltpu.make_async_copy(y_hbm.at[pl.ds(0, B)], rows.at[p], sem).wait()

        def fire_scatter(p):
            for g in range(nsub):
                pltpu.async_copy(rows.at[p, pl.ds(g * G, G)],
                                 acc.at[ridx.at[p, g]], ssem, add=True)

        for ch in range(4):
            kc = 4 * c + ch
            off = kc * N
            for z in range(rows_per_tile // ZR):
                pltpu.sync_copy(zbuf, acc.at[pl.ds(s * rows_per_tile + z * ZR, ZR)])
            plsc.subcore_barrier()

            stage(0, 0, off)

            def pair_body(t, carry):
                for p in range(2):
                    b = 2 * t + p
                    q = 1 - p
                    if p == 0:
                        @pl.when(t > 0)
                        def _():
                            drain(ssem, q)
                    else:
                        drain(ssem, q)
                    stage(b + 1, q, off)
                    drain(gsem, p)
                    multiply(p)
                    fire_scatter(p)
                return carry

            lax.fori_loop(0, NB // 2, pair_body, 0)
            drain(ssem, 1)
            drain(gsem, 0)
            plsc.subcore_barrier()
            pltpu.sync_copy(
                acc.at[pl.ds(s * rows_per_tile, rows_per_tile)],
                out_hbm.at[pl.ds(kc * M + s * rows_per_tile, rows_per_tile)])

    return kfn(col2, row2, vals, y_flat)


def kernel(index, values, y):
    N, K = y.shape
    M = N
    nnz = values.shape[0]
    assert K == 8 * KC

    nnz_pad = ((nnz + 2 * NS * B - 1) // (2 * NS * B)) * (2 * NS * B)
    pad = nnz_pad + B - nnz
    row = jnp.concatenate([index[0], jnp.zeros((pad,), jnp.int32)])
    col = jnp.concatenate([index[1], jnp.zeros((pad,), jnp.int32)])
    vals = jnp.concatenate([values, jnp.zeros((pad,), jnp.float32)])
    vals_rep = jnp.broadcast_to(vals[:, None], (vals.shape[0], 16))
    col2 = col.reshape((nnz_pad + B) // G, G)
    row2 = row.reshape((nnz_pad + B) // G, G)

    y_flat = y.reshape(N, 8, KC).transpose(1, 0, 2).reshape(8 * N, KC)

    NB = nnz_pad // (NS * B)
    out_flat = _sc_spmm(col2, row2, vals_rep, y_flat, M=M, N=N, NB=NB)
    return out_flat.reshape(8, M, KC).transpose(1, 0, 2).reshape(M, K)


# speedup vs baseline: 1.8892x; 1.8892x over previous
import functools

import jax
import jax.numpy as jnp
from jax import lax
from jax.experimental import pallas as pl
from jax.experimental.pallas import tpu as pltpu
from jax.experimental.pallas import tpu_sc as plsc

L = 16
NS = 16
NC = 2
KC = 32
B = 1024
G = 128
ZR = 128


def _sc_spmm(col2, row2, vals, y_flat, *, M, N, NB):
    mesh = plsc.VectorSubcoreMesh(core_axis_name="c", subcore_axis_name="s")
    nsub = B // G

    @functools.partial(
        pl.kernel,
        out_type=jax.ShapeDtypeStruct((8 * M, KC), jnp.float32),
        mesh=mesh,
        scratch_types=[
            pltpu.VMEM_SHARED((M, KC), jnp.float32),
            pltpu.VMEM((2, nsub, G), jnp.int32),
            pltpu.VMEM((2, nsub, G), jnp.int32),
            pltpu.VMEM((2, B), jnp.float32),
            pltpu.VMEM((2, B, KC), jnp.float32),
            pltpu.VMEM((ZR, KC), jnp.float32),
            pltpu.SemaphoreType.DMA,
            pltpu.SemaphoreType.DMA,
        ],
        compiler_params=pltpu.CompilerParams(use_tc_tiling_on_sc=False),
    )
    def kfn(col_hbm, row_hbm, vals_hbm, y_hbm, out_hbm,
            acc, cidx, ridx, vals_v, rows, zbuf, gsem, ssem):
        c = lax.axis_index("c")
        s = lax.axis_index("s")
        zero16 = jnp.zeros((L,), jnp.float32)

        def zb_body(i, carry):
            for j in range(KC // L):
                zbuf[i, pl.ds(L * j, L)] = zero16
            return carry

        lax.fori_loop(0, ZR, zb_body, 0)

        rows_per_tile = M // NS

        def stage(b, p, off):
            base = s * NB + b
            pltpu.sync_copy(col_hbm.at[pl.ds(base * nsub, nsub)], cidx.at[p])
            for r in range(nsub):
                for l in range(G // L):
                    cidx[p, r, pl.ds(L * l, L)] = (
                        cidx[p, r, pl.ds(L * l, L)] + off)
            for g in range(nsub):
                pltpu.async_copy(y_hbm.at[cidx.at[p, g]],
                                 rows.at[p, pl.ds(g * G, G)], gsem)
            pltpu.sync_copy(row_hbm.at[pl.ds(base * nsub, nsub)], ridx.at[p])
            pltpu.sync_copy(vals_hbm.at[pl.ds(base * B, B)], vals_v.at[p])

        def multiply(p):
            @plsc.parallel_loop(0, B // L, unroll=4)
            def mul_body(i):
                v16 = vals_v[p, pl.ds(L * i, L)]
                for l in range(L):
                    v = v16[l]
                    for j in range(KC // L):
                        rows[p, L * i + l, pl.ds(L * j, L)] = (
                            rows[p, L * i + l, pl.ds(L * j, L)] * v)

        def drain(sem, p):
            pltpu.make_async_copy(y_hbm.at[pl.ds(0, B)], rows.at[p], sem).wait()

        def fire_scatter(p):
            for g in range(nsub):
                pltpu.async_copy(rows.at[p, pl.ds(g * G, G)],
                                 acc.at[ridx.at[p, g]], ssem, add=True)

        for ch in range(4):
            kc = 4 * c + ch
            off = kc * N
            for z in range(rows_per_tile // ZR):
                pltpu.sync_copy(zbuf, acc.at[pl.ds(s * rows_per_tile + z * ZR, ZR)])
            plsc.subcore_barrier()

            stage(0, 0, off)

            def pair_body(t, carry):
                for p in range(2):
                    b = 2 * t + p
                    q = 1 - p
                    if p == 0:
                        @pl.when(t > 0)
                        def _():
                            drain(ssem, q)
                    else:
                        drain(ssem, q)
                    stage(b + 1, q, off)
                    drain(gsem, p)
                    multiply(p)
                    fire_scatter(p)
                return carry

            lax.fori_loop(0, NB // 2, pair_body, 0)
            drain(ssem, 1)
            drain(gsem, 0)
            plsc.subcore_barrier()
            pltpu.sync_copy(
                acc.at[pl.ds(s * rows_per_tile, rows_per_tile)],
                out_hbm.at[pl.ds(kc * M + s * rows_per_tile, rows_per_tile)])

    return kfn(col2, row2, vals, y_flat)


def kernel(index, values, y):
    N, K = y.shape
    M = N
    nnz = values.shape[0]
    assert K == 8 * KC

    nnz_pad = ((nnz + 2 * NS * B - 1) // (2 * NS * B)) * (2 * NS * B)
    pad = nnz_pad + B - nnz
    row = jnp.concatenate([index[0], jnp.zeros((pad,), jnp.int32)])
    col = jnp.concatenate([index[1], jnp.zeros((pad,), jnp.int32)])
    vals = jnp.concatenate([values, jnp.zeros((pad,), jnp.float32)])
    col2 = col.reshape((nnz_pad + B) // G, G)
    row2 = row.reshape((nnz_pad + B) // G, G)

    y_flat = y.reshape(N, 8, KC).transpose(1, 0, 2).reshape(8 * N, KC)

    NB = nnz_pad // (NS * B)
    out_flat = _sc_spmm(col2, row2, vals, y_flat, M=M, N=N, NB=NB)
    return out_flat.reshape(8, M, KC).transpose(1, 0, 2).reshape(M, K)


# speedup vs baseline: 1.9222x; 1.0175x over previous
import functools

import jax
import jax.numpy as jnp
from jax import lax
from jax.experimental import pallas as pl
from jax.experimental.pallas import tpu as pltpu
from jax.experimental.pallas import tpu_sc as plsc

L = 16
NS = 16
NC = 2
KC = 32
B = 1024
G = 128
ZR = 128


def _sc_spmm(col2, row2, vals, y_flat, *, M, N, NB):
    mesh = plsc.VectorSubcoreMesh(core_axis_name="c", subcore_axis_name="s")
    nsub = B // G

    @functools.partial(
        pl.kernel,
        out_type=jax.ShapeDtypeStruct((8 * M, KC), jnp.float32),
        mesh=mesh,
        scratch_types=[
            pltpu.VMEM_SHARED((M, KC), jnp.float32),
            pltpu.VMEM((2, nsub, G), jnp.int32),
            pltpu.VMEM((2, nsub, G), jnp.int32),
            pltpu.VMEM((2, B), jnp.float32),
            pltpu.VMEM((2, B, KC), jnp.float32),
            pltpu.VMEM((ZR, KC), jnp.float32),
            pltpu.SemaphoreType.DMA,
            pltpu.SemaphoreType.DMA,
        ],
        compiler_params=pltpu.CompilerParams(use_tc_tiling_on_sc=False),
    )
    def kfn(col_hbm, row_hbm, vals_hbm, y_hbm, out_hbm,
            acc, cidx, ridx, vals_v, rows, zbuf, gsem, ssem):
        c = lax.axis_index("c")
        s = lax.axis_index("s")
        zero16 = jnp.zeros((L,), jnp.float32)

        def zb_body(i, carry):
            for j in range(KC // L):
                zbuf[i, pl.ds(L * j, L)] = zero16
            return carry

        lax.fori_loop(0, ZR, zb_body, 0)

        rows_per_tile = M // NS

        def stage(b, p, off):
            base = s * NB + b
            pltpu.sync_copy(col_hbm.at[pl.ds(base * nsub, nsub)], cidx.at[p])
            for r in range(nsub):
                for l in range(G // L):
                    cidx[p, r, pl.ds(L * l, L)] = (
                        cidx[p, r, pl.ds(L * l, L)] + off)
            for g in range(nsub):
                pltpu.async_copy(y_hbm.at[cidx.at[p, g]],
                                 rows.at[p, pl.ds(g * G, G)], gsem)
            pltpu.sync_copy(row_hbm.at[pl.ds(base * nsub, nsub)], ridx.at[p])
            pltpu.sync_copy(vals_hbm.at[pl.ds(base * B, B)], vals_v.at[p])

        def multiply(p):
            def mul_body(i, carry):
                v16 = vals_v[p, pl.ds(L * i, L)]
                for l in range(L):
                    v = v16[l]
                    for j in range(KC // L):
                        rows[p, L * i + l, pl.ds(L * j, L)] = (
                            rows[p, L * i + l, pl.ds(L * j, L)] * v)
                return carry

            lax.fori_loop(0, B // L, mul_body, 0)

        def drain(sem, p):
            pltpu.make_async_copy(y_hbm.at[pl.ds(0, B)], rows.at[p], sem).wait()

        def fire_scatter(p):
            for g in range(nsub):
                pltpu.async_copy(rows.at[p, pl.ds(g * G, G)],
                                 acc.at[ridx.at[p, g]], ssem, add=True)

        for ch in range(4):
            kc = 4 * c + ch
            off = kc * N
            for z in range(rows_per_tile // ZR):
                pltpu.sync_copy(zbuf, acc.at[pl.ds(s * rows_per_tile + z * ZR, ZR)])
            plsc.subcore_barrier()

            stage(0, 0, off)

            def pair_body(t, carry):
                for p in range(2):
                    b = 2 * t + p
                    q = 1 - p
                    if p == 0:
                        @pl.when(t > 0)
                        def _():
                            drain(ssem, q)
                    else:
                        drain(ssem, q)
                    stage(b + 1, q, off)
                    drain(gsem, p)
                    multiply(p)
                    fire_scatter(p)
                return carry

            lax.fori_loop(0, NB // 2, pair_body, 0)
            drain(ssem, 1)
            drain(gsem, 0)
            plsc.subcore_barrier()
            pltpu.sync_copy(
                acc.at[pl.ds(s * rows_per_tile, rows_per_tile)],
                out_hbm.at[pl.ds(kc * M + s * rows_per_tile, rows_per_tile)])

    return kfn(col2, row2, vals, y_flat)


def kernel(index, values, y):
    N, K = y.shape
    M = N
    nnz = values.shape[0]
    assert K == 8 * KC

    nnz_pad = ((nnz + 2 * NS * B - 1) // (2 * NS * B)) * (2 * NS * B)
    pad = nnz_pad + B - nnz
    row = jnp.concatenate([index[0], jnp.zeros((pad,), jnp.int32)])
    col = jnp.concatenate([index[1], jnp.zeros((pad,), jnp.int32)])
    vals = jnp.concatenate([values, jnp.zeros((pad,), jnp.float32)])
    col2 = col.reshape((nnz_pad + B) // G, G)
    row2 = row.reshape((nnz_pad + B) // G, G)

    y_flat = y.reshape(N, 8, KC).transpose(1, 0, 2).reshape(8 * N, KC)

    NB = nnz_pad // (NS * B)
    out_flat = _sc_spmm(col2, row2, vals, y_flat, M=M, N=N, NB=NB)
    return out_flat.reshape(8, M, KC).transpose(1, 0, 2).reshape(M, K)
